# baseline (speedup 1.0000x reference)
import functools

import jax
import jax.numpy as jnp
from jax import lax
from jax.experimental import pallas as pl
from jax.experimental.pallas import tpu as pltpu
from jax.experimental.pallas import tpu_sc as plsc

_LATENT_DIM = 256
_CODE_SIZE = 1024
_BETA = 0.25

_TM = 2048
_CHUNKS = (8192, 8192, 2048)

_NC = 2
_NS = 16
_NL = 16
_NW = _NC * _NS
_CHUNK = 64


_H = 8


def _tc_body(x_ref, cb_ref, idx_ref, loss_ref, cn_ref, *, n_total):
    i = pl.program_id(0)
    cb = cb_ref[...]
    iota_f = lax.broadcasted_iota(jnp.int32, (1, _CODE_SIZE), 1).astype(
        jnp.float32)
    hm = _TM // _H
    part = jnp.zeros((), jnp.float32)
    for h in range(_H):
        x = x_ref[pl.ds(h * hm, hm), :]
        scores = lax.dot_general(
            x, cb, (((1,), (1,)), ((), ())),
            preferred_element_type=jnp.float32,
            precision=lax.Precision.DEFAULT,
        )
        minval = jnp.min(scores, axis=1, keepdims=True)
        idxf = jnp.min(jnp.where(scores == minval, iota_f,
                                 float(_CODE_SIZE)), axis=1)
        idx_ref[pl.ds(h * hm, hm)] = idxf.astype(jnp.int32)
        part = part + (jnp.sum(x * x) - 2.0 * jnp.sum(minval))

    @pl.when(i == 0)
    def _cn():
        cn_ref[...] = lax.dot_general(
            jnp.ones((1, cb.shape[1]), jnp.float32), cb * cb,
            (((1,), (1,)), ((), ())),
            preferred_element_type=jnp.float32,
        )

    @pl.when(i == 0)
    def _init():
        loss_ref[...] = jnp.zeros_like(loss_ref)

    loss_ref[...] += part.reshape(1, 1)

    @pl.when(i == pl.num_programs(0) - 1)
    def _finish():
        loss_ref[...] = loss_ref[...] * ((1.0 + _BETA) / n_total)


def _sc_gather_body(cb_hbm, idx_hbm, cn_hbm, q_hbm, cnp_hbm,
                    idx_v, rows_v, cn_v, acc_v, sem, sem2, *, chunk_base):
    wid = lax.axis_index("s") * _NC + lax.axis_index("c")
    b_per_w = idx_v.shape[0]
    pltpu.sync_copy(idx_hbm.at[pl.ds(wid * b_per_w, b_per_w)], idx_v)
    base = chunk_base + wid * b_per_w
    acc = jnp.zeros((_NL,), jnp.float32)
    for j in range(b_per_w // _CHUNK):
        idx_c = idx_v.at[pl.ds(j * _CHUNK, _CHUNK)]
        row_dma = pltpu.async_copy(cb_hbm.at[idx_c], rows_v, sem)
        cn_dma = pltpu.async_copy(cn_hbm.at[idx_c], cn_v, sem2)
        row_dma.wait()
        cn_dma.wait()
        pltpu.sync_copy(rows_v, q_hbm.at[pl.ds(base + j * _CHUNK, _CHUNK)])
        for k in range(_CHUNK // _NL):
            acc = acc + cn_v[pl.ds(k * _NL, _NL)]
    acc_v[...] = acc
    pltpu.sync_copy(acc_v, cnp_hbm.at[wid])


def kernel(x, codebook):
    b, t, d = x.shape
    m = b * t
    xf = x.reshape(m, d)

    q_ref = jax.empty_ref(jax.ShapeDtypeStruct((m, d), jnp.float32))
    mesh = plsc.VectorSubcoreMesh(core_axis_name="c", subcore_axis_name="s")

    idx_parts, loss_parts, cnp_parts = [], [], []
    cn_flat = None
    offset = 0
    for mc in _CHUNKS:
        steps = mc // _TM
        b_per_w = mc // _NW
        base_step = offset // _TM
        x_map = functools.partial(lambda i, s: (s + i, 0), s=base_step)
        idx_c, loss_c, cn_c = pl.pallas_call(
            functools.partial(_tc_body, n_total=float(m * d)),
            grid=(steps,),
            in_specs=[
                pl.BlockSpec((_TM, d), x_map),
                pl.BlockSpec((_CODE_SIZE, d), lambda i: (0, 0)),
            ],
            out_specs=[
                pl.BlockSpec((_TM,), lambda i: (i,)),
                pl.BlockSpec((1, 1), lambda i: (0, 0)),
                pl.BlockSpec((1, _CODE_SIZE), lambda i: (0, 0)),
            ],
            out_shape=[
                jax.ShapeDtypeStruct((mc,), jnp.int32),
                jax.ShapeDtypeStruct((1, 1), jnp.float32),
                jax.ShapeDtypeStruct((1, _CODE_SIZE), jnp.float32),
            ],
        )(xf, codebook)
        if cn_flat is None:
            cn_flat = cn_c.reshape(_CODE_SIZE)
        sc_gather = functools.partial(
            pl.kernel,
            out_type=jax.ShapeDtypeStruct((_NW, _NL), jnp.float32),
            mesh=mesh,
            scratch_types=[
                pltpu.VMEM((b_per_w,), jnp.int32),
                pltpu.VMEM((_CHUNK, d), jnp.float32),
                pltpu.VMEM((_CHUNK,), jnp.float32),
                pltpu.VMEM((_NL,), jnp.float32),
                pltpu.SemaphoreType.DMA,
                pltpu.SemaphoreType.DMA,
            ],
        )(functools.partial(_sc_gather_body, chunk_base=offset))
        cnp_c = sc_gather(codebook, idx_c, cn_flat, q_ref)
        idx_parts.append(idx_c)
        loss_parts.append(loss_c.reshape(()))
        cnp_parts.append(cnp_c)
        offset += mc

    q = jax.freeze(q_ref)
    idx = jnp.concatenate(idx_parts)
    loss = (sum(loss_parts)
            + ((1.0 + _BETA) / float(m * d)) * sum(jnp.sum(p) for p in cnp_parts))
    return (q.reshape(b, t, d), loss, idx.reshape(b, t))


# speedup vs baseline: 1.0512x; 1.0512x over previous
import functools

import jax
import jax.numpy as jnp
from jax import lax
from jax.experimental import pallas as pl
from jax.experimental.pallas import tpu as pltpu
from jax.experimental.pallas import tpu_sc as plsc

_LATENT_DIM = 256
_CODE_SIZE = 1024
_BETA = 0.25

_TM = 2048
_K = 3

_NC = 2
_NS = 16
_NL = 16
_NW = _NC * _NS
_CHUNK = 96


_H = 8


def _tc_body(x_ref, cb_ref, idx_ref, loss_ref, cn_ref, *, n_total):
    i = pl.program_id(0)
    cb = cb_ref[...]
    iota_f = lax.broadcasted_iota(jnp.int32, (1, _CODE_SIZE), 1).astype(
        jnp.float32)
    hm = _TM // _H
    part = jnp.zeros((), jnp.float32)
    for h in range(_H):
        x = x_ref[pl.ds(h * hm, hm), :]
        scores = lax.dot_general(
            x, cb, (((1,), (1,)), ((), ())),
            preferred_element_type=jnp.float32,
            precision=lax.Precision.DEFAULT,
        )
        minval = jnp.min(scores, axis=1, keepdims=True)
        idxf = jnp.min(jnp.where(scores == minval, iota_f,
                                 float(_CODE_SIZE)), axis=1)
        idx_ref[pl.ds(h * hm, hm)] = idxf.astype(jnp.int32)
        part = part + (jnp.sum(x * x) - 2.0 * jnp.sum(minval))

    @pl.when(i == 0)
    def _cn():
        cn_ref[...] = lax.dot_general(
            jnp.ones((1, cb.shape[1]), jnp.float32), cb * cb,
            (((1,), (1,)), ((), ())),
            preferred_element_type=jnp.float32,
        )

    @pl.when(i == 0)
    def _init():
        loss_ref[...] = jnp.zeros_like(loss_ref)

    loss_ref[...] += part.reshape(1, 1)

    @pl.when(i == pl.num_programs(0) - 1)
    def _finish():
        loss_ref[...] = loss_ref[...] * ((1.0 + _BETA) / n_total)


def _sc_gather_body(cb_hbm, idx_hbm, cn_hbm, q_hbm, cnp_hbm,
                    idx_v, rows_v, cn_v, acc_v, sem, sem2, *, chunk_base):
    wid = lax.axis_index("s") * _NC + lax.axis_index("c")
    b_per_w = idx_v.shape[0]
    pltpu.sync_copy(idx_hbm.at[pl.ds(wid * b_per_w, b_per_w)], idx_v)
    base = chunk_base + wid * b_per_w
    acc = jnp.zeros((_NL,), jnp.float32)
    for j in range(b_per_w // _CHUNK):
        idx_c = idx_v.at[pl.ds(j * _CHUNK, _CHUNK)]
        row_dma = pltpu.async_copy(cb_hbm.at[idx_c], rows_v, sem)
        cn_dma = pltpu.async_copy(cn_hbm.at[idx_c], cn_v, sem2)
        row_dma.wait()
        cn_dma.wait()
        pltpu.sync_copy(rows_v, q_hbm.at[pl.ds(base + j * _CHUNK, _CHUNK)])
        for k in range(_CHUNK // _NL):
            acc = acc + cn_v[pl.ds(k * _NL, _NL)]
    acc_v[...] = acc
    pltpu.sync_copy(acc_v, cnp_hbm.at[wid])


def kernel(x, codebook):
    b, t, d = x.shape
    m = b * t
    xf = x.reshape(m, d)
    mc = m // _K
    steps = mc // _TM
    b_per_w = mc // _NW

    q_ref = jax.empty_ref(jax.ShapeDtypeStruct((m, d), jnp.float32))
    mesh = plsc.VectorSubcoreMesh(core_axis_name="c", subcore_axis_name="s")

    idx_parts, loss_parts, cnp_parts = [], [], []
    cn_flat = None
    for c in range(_K):
        x_map = functools.partial(lambda i, c: (c * steps + i, 0), c=c)
        idx_c, loss_c, cn_c = pl.pallas_call(
            functools.partial(_tc_body, n_total=float(m * d)),
            grid=(steps,),
            in_specs=[
                pl.BlockSpec((_TM, d), x_map),
                pl.BlockSpec((_CODE_SIZE, d), lambda i: (0, 0)),
            ],
            out_specs=[
                pl.BlockSpec((_TM,), lambda i: (i,)),
                pl.BlockSpec((1, 1), lambda i: (0, 0)),
                pl.BlockSpec((1, _CODE_SIZE), lambda i: (0, 0)),
            ],
            out_shape=[
                jax.ShapeDtypeStruct((mc,), jnp.int32),
                jax.ShapeDtypeStruct((1, 1), jnp.float32),
                jax.ShapeDtypeStruct((1, _CODE_SIZE), jnp.float32),
            ],
        )(xf, codebook)
        if cn_flat is None:
            cn_flat = cn_c.reshape(_CODE_SIZE)
        sc_gather = functools.partial(
            pl.kernel,
            out_type=jax.ShapeDtypeStruct((_NW, _NL), jnp.float32),
            mesh=mesh,
            scratch_types=[
                pltpu.VMEM((b_per_w,), jnp.int32),
                pltpu.VMEM((_CHUNK, d), jnp.float32),
                pltpu.VMEM((_CHUNK,), jnp.float32),
                pltpu.VMEM((_NL,), jnp.float32),
                pltpu.SemaphoreType.DMA,
                pltpu.SemaphoreType.DMA,
            ],
        )(functools.partial(_sc_gather_body, chunk_base=c * mc))
        cnp_c = sc_gather(codebook, idx_c, cn_flat, q_ref)
        idx_parts.append(idx_c)
        loss_parts.append(loss_c.reshape(()))
        cnp_parts.append(cnp_c)

    q = jax.freeze(q_ref)
    idx = jnp.concatenate(idx_parts)
    loss = (sum(loss_parts)
            + ((1.0 + _BETA) / float(m * d)) * sum(jnp.sum(p) for p in cnp_parts))
    return (q.reshape(b, t, d), loss, idx.reshape(b, t))
